# baseline (speedup 1.0000x reference)
import functools

import jax
import jax.numpy as jnp
from jax import lax
from jax.experimental import pallas as pl
from jax.experimental.pallas import tpu as pltpu
from jax.experimental.pallas import tpu_sc as plsc

N = 262144
C = 256
CH = 128

_info = plsc.get_sparse_core_info()
_NC, _NS, _L = _info.num_cores, _info.num_subcores, _info.num_lanes
NW = _NC * _NS
ROWS_W = N // NW
CHUNKS = ROWS_W // CH


@functools.partial(
    pl.kernel,
    mesh=plsc.VectorSubcoreMesh(core_axis_name="c", subcore_axis_name="s"),
    out_type=jax.ShapeDtypeStruct((N, C), jnp.float32),
    compiler_params=pltpu.CompilerParams(needs_layout_passes=False),
    scratch_types=[
        pltpu.VMEM((ROWS_W,), jnp.int32),
        pltpu.VMEM((_L,), jnp.float32),
        pltpu.VMEM((CH, C), jnp.float32),
        pltpu.VMEM((CH, C), jnp.float32),
        pltpu.VMEM((CH, C), jnp.float32),
        pltpu.SemaphoreType.DMA,
        pltpu.SemaphoreType.DMA,
        pltpu.SemaphoreType.DMA,
    ],
)
def _onehot_sc(labels_hbm, scale_hbm, out_hbm, lab_v, sc_v, buf0, buf1, buf2,
               sem0, sem1, sem2):
    wid = lax.axis_index("s") * _NC + lax.axis_index("c")
    base = wid * ROWS_W
    pltpu.sync_copy(labels_hbm.at[pl.ds(base, ROWS_W)], lab_v)
    pltpu.sync_copy(scale_hbm, sc_v)
    sv = sc_v[...]
    zv = sv * 0.0
    iota = lax.iota(jnp.int32, _L)

    bufs = (buf0, buf1, buf2)
    sems = (sem0, sem1, sem2)
    nbuf = len(bufs)

    def zero_body(r, carry):
        for j in range(C // _L):
            for buf in bufs:
                buf[r, pl.ds(j * _L, _L)] = zv
        return carry

    lax.fori_loop(0, CH, zero_body, 0)

    def scatter_chunk(buf, chunk, vals):
        off = chunk * CH
        for j in range(CH // _L):
            labs = lab_v[pl.ds(off + j * _L, _L)]
            rows = iota + (j * _L)
            plsc.store_scatter(buf, [rows, labs], vals)

    def dst_ref(chunk):
        return out_hbm.at[pl.ds(base + chunk * CH, CH)]

    def advance(b, c):
        pltpu.make_async_copy(bufs[b], dst_ref(c - nbuf), sems[b]).wait()
        scatter_chunk(bufs[b], c - nbuf, zv)
        scatter_chunk(bufs[b], c, sv)
        pltpu.make_async_copy(bufs[b], dst_ref(c), sems[b]).start()

    for b in range(nbuf):
        scatter_chunk(bufs[b], b, sv)
        pltpu.make_async_copy(bufs[b], dst_ref(b), sems[b]).start()

    main_groups = (CHUNKS - nbuf) // nbuf

    def chunk_body(i, carry):
        c0 = nbuf + nbuf * i
        for b in range(nbuf):
            advance(b, c0 + b)
        return carry

    lax.fori_loop(0, main_groups, chunk_body, 0)

    done = nbuf + main_groups * nbuf
    for c in range(done, CHUNKS):
        advance(c % nbuf, c)
    for b in range(nbuf):
        pltpu.make_async_copy(bufs[b], dst_ref(0), sems[b]).wait()


def kernel(labels, scale):
    scale_vec = jnp.broadcast_to(scale.astype(jnp.float32), (_L,))
    return _onehot_sc(labels, scale_vec)


# speedup vs baseline: 1.0229x; 1.0229x over previous
import functools

import jax
import jax.numpy as jnp
from jax import lax
from jax.experimental import pallas as pl
from jax.experimental.pallas import tpu as pltpu
from jax.experimental.pallas import tpu_sc as plsc

N = 262144
C = 256
CH = 128

_info = plsc.get_sparse_core_info()
_NC, _NS, _L = _info.num_cores, _info.num_subcores, _info.num_lanes
NW = _NC * _NS
ROWS_W = N // NW
CHUNKS = ROWS_W // CH


@functools.partial(
    pl.kernel,
    mesh=plsc.VectorSubcoreMesh(core_axis_name="c", subcore_axis_name="s"),
    out_type=jax.ShapeDtypeStruct((N, C), jnp.float32),
    compiler_params=pltpu.CompilerParams(needs_layout_passes=False),
    scratch_types=[
        pltpu.VMEM((ROWS_W,), jnp.int32),
        pltpu.VMEM((_L,), jnp.float32),
        pltpu.VMEM((CH, C), jnp.float32),
        pltpu.VMEM((CH, C), jnp.float32),
        pltpu.SemaphoreType.DMA,
        pltpu.SemaphoreType.DMA,
    ],
)
def _onehot_sc(labels_hbm, scale_hbm, out_hbm, lab_v, sc_v, buf0, buf1,
               sem0, sem1):
    wid = lax.axis_index("s") * _NC + lax.axis_index("c")
    base = wid * ROWS_W
    pltpu.sync_copy(labels_hbm.at[pl.ds(base, ROWS_W)], lab_v)
    pltpu.sync_copy(scale_hbm, sc_v)
    sv = sc_v[...]
    zv = sv * 0.0
    iota = lax.iota(jnp.int32, _L)

    bufs = (buf0, buf1)
    sems = (sem0, sem1)
    nbuf = len(bufs)

    def zero_body(r, carry):
        for j in range(C // _L):
            for buf in bufs:
                buf[r, pl.ds(j * _L, _L)] = zv
        return carry

    lax.fori_loop(0, CH, zero_body, 0)

    def scatter_chunk(buf, chunk, vals):
        off = chunk * CH
        for j in range(CH // _L):
            labs = lab_v[pl.ds(off + j * _L, _L)]
            rows = iota + (j * _L)
            plsc.store_scatter(buf, [rows, labs], vals)

    def dst_ref(chunk):
        return out_hbm.at[pl.ds(base + chunk * CH, CH)]

    def advance(b, c):
        pltpu.make_async_copy(bufs[b], dst_ref(c - nbuf), sems[b]).wait()
        scatter_chunk(bufs[b], c - nbuf, zv)
        scatter_chunk(bufs[b], c, sv)
        pltpu.make_async_copy(bufs[b], dst_ref(c), sems[b]).start()

    for b in range(nbuf):
        scatter_chunk(bufs[b], b, sv)
        pltpu.make_async_copy(bufs[b], dst_ref(b), sems[b]).start()

    main_groups = (CHUNKS - nbuf) // nbuf

    def chunk_body(i, carry):
        c0 = nbuf + nbuf * i
        for b in range(nbuf):
            advance(b, c0 + b)
        return carry

    lax.fori_loop(0, main_groups, chunk_body, 0)

    done = nbuf + main_groups * nbuf
    for c in range(done, CHUNKS):
        advance(c % nbuf, c)
    for b in range(nbuf):
        pltpu.make_async_copy(bufs[b], dst_ref(0), sems[b]).wait()


def kernel(labels, scale):
    scale_vec = jnp.broadcast_to(scale.astype(jnp.float32), (_L,))
    return _onehot_sc(labels, scale_vec)
